# baseline (speedup 1.0000x reference)
import functools

import jax
import jax.numpy as jnp
from jax import lax
from jax.experimental import pallas as pl
from jax.experimental.pallas import tpu as pltpu
from jax.experimental.pallas import tpu_sc as plsc

N = 10000
E = 320000
D_FEAT = 128
D_EDGE = 16
HID = 64
STEPS = 3

NC = 2
NS = 16
RSTRIDE = 624
RSIZE = 640
EDGES_PER_TILE = E // (NC * NS)
CHUNK = 80
NCHUNKS = EDGES_PER_TILE // CHUNK

LANES = 16

_PERM = []
for _m in range(HID // 32):
    for _i in range(16):
        _PERM += [32 * _m + _i, 32 * _m + 16 + _i]
_PERM = tuple(_PERM)


BLK_N = 1000
BLK_E = 6400


def _embed_body(x_ref, we_ref, be_ref, wmh_ref, bm_ref, h_ref, g_ref):
    h = jnp.dot(x_ref[...], we_ref[...], preferred_element_type=jnp.float32)
    h = h + be_ref[...]
    h_ref[...] = h
    g_ref[...] = (
        jnp.dot(h, wmh_ref[...], preferred_element_type=jnp.float32)
        + bm_ref[...]
    ).astype(jnp.bfloat16)


_embed_call = pl.pallas_call(
    _embed_body,
    grid=(N // BLK_N,),
    in_specs=[
        pl.BlockSpec((BLK_N, D_FEAT), lambda i: (i, 0)),
        pl.BlockSpec((D_FEAT, HID), lambda i: (0, 0)),
        pl.BlockSpec((1, HID), lambda i: (0, 0)),
        pl.BlockSpec((HID, HID), lambda i: (0, 0)),
        pl.BlockSpec((1, HID), lambda i: (0, 0)),
    ],
    out_specs=[
        pl.BlockSpec((BLK_N, HID), lambda i: (i, 0)),
        pl.BlockSpec((BLK_N, HID), lambda i: (i, 0)),
    ],
    out_shape=[
        jax.ShapeDtypeStruct((N, HID), jnp.float32),
        jax.ShapeDtypeStruct((N, HID), jnp.bfloat16),
    ],
)


def _eaproj_body(eat_a_ref, eat_b_ref, wme_ref, out_ref):
    dn = (((0,), (0,)), ((), ()))
    pa = lax.dot_general(eat_a_ref[...], wme_ref[...], dimension_numbers=dn,
                         preferred_element_type=jnp.float32)
    pb = lax.dot_general(eat_b_ref[...], wme_ref[...], dimension_numbers=dn,
                         preferred_element_type=jnp.float32)
    out_ref[...] = jnp.concatenate([pa, pb], axis=1).astype(jnp.bfloat16)


_eaproj_call = pl.pallas_call(
    _eaproj_body,
    grid=(E // 2 // BLK_E,),
    in_specs=[
        pl.BlockSpec((D_EDGE, BLK_E), lambda i: (0, i)),
        pl.BlockSpec((D_EDGE, BLK_E), lambda i: (0, i + E // 2 // BLK_E)),
        pl.BlockSpec((D_EDGE, HID), lambda i: (0, 0)),
    ],
    out_specs=pl.BlockSpec((BLK_E, 2 * HID), lambda i: (i, 0)),
    out_shape=jax.ShapeDtypeStruct((E // 2, 2 * HID), jnp.bfloat16),
)


def _update_body(h_ref, p_ref, wu1_ref, wu2_ref, bu_ref, wmh_ref, bm_ref,
                 h_out_ref, g_out_ref):
    agg = p_ref[0] + p_ref[1]
    h = jnp.dot(h_ref[...], wu1_ref[...], preferred_element_type=jnp.float32)
    h = h + jnp.dot(agg, wu2_ref[...], preferred_element_type=jnp.float32)
    h = jnp.maximum(h + bu_ref[...], 0.0)
    h_out_ref[...] = h
    if g_out_ref is not None:
        g_out_ref[...] = (
            jnp.dot(h, wmh_ref[...], preferred_element_type=jnp.float32)
            + bm_ref[...]
        ).astype(jnp.bfloat16)


def _make_update_call(with_g: bool):
    out_specs = [pl.BlockSpec((BLK_N, HID), lambda i: (i, 0))]
    out_shape = [jax.ShapeDtypeStruct((N, HID), jnp.float32)]
    if with_g:
        out_specs.append(pl.BlockSpec((BLK_N, HID), lambda i: (i, 0)))
        out_shape.append(jax.ShapeDtypeStruct((N, HID), jnp.bfloat16))
    body = _update_body if with_g else (
        lambda h, p, w1, w2, b, wm, bm, ho: _update_body(
            h, p, w1, w2, b, wm, bm, ho, None)
    )
    return pl.pallas_call(
        body,
        grid=(N // BLK_N,),
        in_specs=[
            pl.BlockSpec((BLK_N, HID), lambda i: (i, 0)),
            pl.BlockSpec((NC, BLK_N, HID), lambda i: (0, i, 0)),
            pl.BlockSpec((HID, HID), lambda i: (0, 0)),
            pl.BlockSpec((HID, HID), lambda i: (0, 0)),
            pl.BlockSpec((1, HID), lambda i: (0, 0)),
            pl.BlockSpec((HID, HID), lambda i: (0, 0)),
            pl.BlockSpec((1, HID), lambda i: (0, 0)),
        ],
        out_specs=out_specs,
        out_shape=out_shape,
    )


_update_call = _make_update_call(True)
_update_last_call = _make_update_call(False)


_sc_mesh = plsc.VectorSubcoreMesh(core_axis_name="c", subcore_axis_name="s")


RING = 3
MAIN_ITERS = (NCHUNKS - 2) // RING


@functools.partial(
    pl.kernel,
    out_type=jax.ShapeDtypeStruct((NC, N, HID), jnp.float32),
    mesh=_sc_mesh,
    compiler_params=pltpu.CompilerParams(use_tc_tiling_on_sc=False,
                                         needs_layout_passes=False),
    scratch_types=[
        pltpu.VMEM((RING, CHUNK), jnp.int32),
        pltpu.VMEM((RING, CHUNK), jnp.int32),
        pltpu.VMEM((RING, CHUNK, HID), jnp.bfloat16),
        pltpu.VMEM((RING, CHUNK, HID), jnp.bfloat16),
        pltpu.VMEM((RING, CHUNK, HID), jnp.float32),
        pltpu.VMEM_SHARED((N, HID), jnp.bfloat16),
        pltpu.VMEM_SHARED((N, HID), jnp.float32),
        pltpu.SemaphoreType.DMA((RING,)),
        pltpu.SemaphoreType.DMA((RING,)),
        pltpu.SemaphoreType.DMA((RING,)),
    ],
)
def _edge_pass(g_hbm, ea_hbm, src_hbm, dst_hbm, out_hbm,
               idx_s, idx_d, rows, ea_buf, msg, g_sp, acc_sp,
               load_sem, gather_sem, scat_sem):
    c = lax.axis_index("c")
    s = lax.axis_index("s")
    row0 = s * RSTRIDE
    ebase = (c * NS + s) * EDGES_PER_TILE
    eabase = s * EDGES_PER_TILE

    def issue_loads(j, b):
        base = ebase + j * CHUNK
        erow = eabase + j * CHUNK

        @pl.when(c == 0)
        def _():
            pltpu.async_copy(ea_hbm.at[pl.ds(erow, CHUNK), pl.ds(0, HID)],
                             ea_buf.at[b], load_sem.at[b])

        @pl.when(c == 1)
        def _():
            pltpu.async_copy(ea_hbm.at[pl.ds(erow, CHUNK), pl.ds(HID, HID)],
                             ea_buf.at[b], load_sem.at[b])

        pltpu.async_copy(src_hbm.at[pl.ds(base, CHUNK)], idx_s.at[b],
                         load_sem.at[b])
        pltpu.async_copy(dst_hbm.at[pl.ds(base, CHUNK)], idx_d.at[b],
                         load_sem.at[b])

    def wait_loads(b):
        pltpu.make_async_copy(src_hbm.at[pl.ds(0, CHUNK)], idx_s.at[b],
                              load_sem.at[b]).wait()
        pltpu.make_async_copy(dst_hbm.at[pl.ds(0, CHUNK)], idx_d.at[b],
                              load_sem.at[b]).wait()
        pltpu.make_async_copy(ea_hbm.at[pl.ds(0, CHUNK), pl.ds(0, HID)],
                              ea_buf.at[b], load_sem.at[b]).wait()

    def issue_gather(b):
        pltpu.async_copy(g_sp.at[idx_s.at[b]], rows.at[b], gather_sem.at[b])

    def wait_gather(b):
        pltpu.make_async_copy(g_sp.at[idx_s.at[b]], rows.at[b],
                              gather_sem.at[b]).wait()

    def issue_scat(b):
        pltpu.async_copy(msg.at[b], acc_sp.at[idx_d.at[b]], scat_sem.at[b],
                         add=True)

    def wait_scat(b):
        pltpu.make_async_copy(msg.at[b], acc_sp.at[idx_d.at[b]],
                              scat_sem.at[b]).wait()

    UNROLL = 8

    def compute(b):
        def relu_body(jo, inner):
            j0 = jo * UNROLL
            for jj in range(UNROLL):
                for m in range(HID // 32):
                    sl32 = pl.ds(m * 32, 32)
                    s_bf = jnp.maximum(
                        rows[b, j0 + jj, sl32] + ea_buf[b, j0 + jj, sl32],
                        jnp.bfloat16(0.0))
                    a0, a1 = plsc.unpack(s_bf,
                                         format=plsc.PackFormat.INTERLEAVED)
                    msg[b, j0 + jj, pl.ds(m * 32, LANES)] = a0
                    msg[b, j0 + jj, pl.ds(m * 32 + LANES, LANES)] = a1
            return inner

        lax.fori_loop(0, CHUNK // UNROLL, relu_body, 0)

    issue_loads(0, 0)
    issue_loads(1, 1)
    for i in range(RSIZE // CHUNK):
        sl = pl.ds(row0 + i * CHUNK, CHUNK)
        pltpu.sync_copy(g_hbm.at[sl], rows.at[2])
        pltpu.sync_copy(rows.at[2], g_sp.at[sl])
    zero = jnp.zeros((LANES,), jnp.float32)

    def zero_body(j, carry):
        for k in range(HID // LANES):
            msg[2, j, pl.ds(k * LANES, LANES)] = zero
        return carry

    lax.fori_loop(0, CHUNK, zero_body, 0)
    for i in range(RSIZE // CHUNK):
        pltpu.sync_copy(msg.at[2], acc_sp.at[pl.ds(row0 + i * CHUNK, CHUNK)])
    plsc.subcore_barrier()
    wait_loads(0)
    issue_gather(0)

    def outer(i2, carry):
        for b in range(RING):
            j = RING * i2 + b
            bn = (b + 1) % RING
            bp = (b + 2) % RING
            if b == 0:

                @pl.when(i2 > 0)
                def _():
                    wait_scat(bp)
            else:
                wait_scat(bp)
            issue_loads(j + 2, bp)
            wait_loads(bn)
            issue_gather(bn)
            wait_gather(b)
            compute(b)
            issue_scat(b)
        return carry

    lax.fori_loop(0, MAIN_ITERS, outer, 0)

    wait_scat(2)
    wait_loads(1)
    issue_gather(1)
    wait_gather(0)
    compute(0)
    issue_scat(0)
    wait_gather(1)
    compute(1)
    issue_scat(1)
    wait_scat(0)
    wait_scat(1)
    plsc.subcore_barrier()

    for i in range(RSIZE // CHUNK):
        b = i % 2
        sl = pl.ds(row0 + i * CHUNK, CHUNK)
        pltpu.sync_copy(acc_sp.at[sl], msg.at[b])
        pltpu.sync_copy(msg.at[b], out_hbm.at[c, sl])


@jax.jit
def kernel(x, edge_index, edge_attr, W_embed, b_embed, W_msg, b_msg, W_upd,
           b_upd):
    src = edge_index[0].astype(jnp.int32)
    dst = edge_index[1].astype(jnp.int32)
    perm = jnp.array(_PERM, dtype=jnp.int32)
    wmh = W_msg[:HID, perm]
    wme = W_msg[HID:, perm]
    wu1 = W_upd[:HID]
    wu2 = W_upd[HID:]
    bm = b_msg[perm].reshape(1, HID)
    be = b_embed.reshape(1, HID)
    bu = b_upd.reshape(1, HID)

    h, g = _embed_call(x, W_embed, be, wmh, bm)
    eat = edge_attr.T
    ea = _eaproj_call(eat, eat, wme)
    for t in range(STEPS):
        parts = _edge_pass(g, ea, src, dst)
        if t < STEPS - 1:
            h, g = _update_call(h, parts, wu1, wu2, bu, wmh, bm)
        else:
            h = _update_last_call(h, parts, wu1, wu2, bu, wmh, bm)[0]
    return h


# speedup vs baseline: 2.3457x; 2.3457x over previous
import functools

import jax
import jax.numpy as jnp
from jax import lax
from jax.experimental import pallas as pl
from jax.experimental.pallas import tpu as pltpu
from jax.experimental.pallas import tpu_sc as plsc

N = 10000
E = 320000
D_FEAT = 128
D_EDGE = 16
HID = 64
STEPS = 3

NC = 2
NS = 16
RSTRIDE = 624
RSIZE = 640
EDGES_PER_TILE = E // (NC * NS)
CHUNK = 80
NCHUNKS = EDGES_PER_TILE // CHUNK

LANES = 16

_PERM = []
for _m in range(HID // 32):
    for _i in range(16):
        _PERM += [32 * _m + _i, 32 * _m + 16 + _i]
_PERM = tuple(_PERM)


BLK_N = 1000
BLK_E = 6400


def _embed_body(x_ref, we_ref, be_ref, wmh_ref, bm_ref, h_ref, g_ref):
    h = jnp.dot(x_ref[...], we_ref[...], preferred_element_type=jnp.float32)
    h = h + be_ref[...]
    h_ref[...] = h
    g_ref[...] = (
        jnp.dot(h, wmh_ref[...], preferred_element_type=jnp.float32)
        + bm_ref[...]
    )


_embed_call = pl.pallas_call(
    _embed_body,
    grid=(N // BLK_N,),
    in_specs=[
        pl.BlockSpec((BLK_N, D_FEAT), lambda i: (i, 0)),
        pl.BlockSpec((D_FEAT, HID), lambda i: (0, 0)),
        pl.BlockSpec((1, HID), lambda i: (0, 0)),
        pl.BlockSpec((HID, HID), lambda i: (0, 0)),
        pl.BlockSpec((1, HID), lambda i: (0, 0)),
    ],
    out_specs=[
        pl.BlockSpec((BLK_N, HID), lambda i: (i, 0)),
        pl.BlockSpec((BLK_N, HID), lambda i: (i, 0)),
    ],
    out_shape=[
        jax.ShapeDtypeStruct((N, HID), jnp.float32),
        jax.ShapeDtypeStruct((N, HID), jnp.float32),
    ],
)


def _eaproj_body(eat_a_ref, eat_b_ref, wme_ref, out_ref):
    dn = (((0,), (0,)), ((), ()))
    pa = lax.dot_general(eat_a_ref[...], wme_ref[...], dimension_numbers=dn,
                         preferred_element_type=jnp.float32)
    pb = lax.dot_general(eat_b_ref[...], wme_ref[...], dimension_numbers=dn,
                         preferred_element_type=jnp.float32)
    out_ref[...] = jnp.concatenate([pa, pb], axis=1)


_eaproj_call = pl.pallas_call(
    _eaproj_body,
    grid=(E // 2 // BLK_E,),
    in_specs=[
        pl.BlockSpec((D_EDGE, BLK_E), lambda i: (0, i)),
        pl.BlockSpec((D_EDGE, BLK_E), lambda i: (0, i + E // 2 // BLK_E)),
        pl.BlockSpec((D_EDGE, HID), lambda i: (0, 0)),
    ],
    out_specs=pl.BlockSpec((BLK_E, 2 * HID), lambda i: (i, 0)),
    out_shape=jax.ShapeDtypeStruct((E // 2, 2 * HID), jnp.float32),
)


def _update_body(h_ref, p_ref, wu1_ref, wu2_ref, bu_ref, wmh_ref, bm_ref,
                 h_out_ref, g_out_ref):
    agg = p_ref[0] + p_ref[1]
    h = jnp.dot(h_ref[...], wu1_ref[...], preferred_element_type=jnp.float32)
    h = h + jnp.dot(agg, wu2_ref[...], preferred_element_type=jnp.float32)
    h = jnp.maximum(h + bu_ref[...], 0.0)
    h_out_ref[...] = h
    if g_out_ref is not None:
        g_out_ref[...] = (
            jnp.dot(h, wmh_ref[...], preferred_element_type=jnp.float32)
            + bm_ref[...]
        )


def _make_update_call(with_g: bool):
    out_specs = [pl.BlockSpec((BLK_N, HID), lambda i: (i, 0))]
    out_shape = [jax.ShapeDtypeStruct((N, HID), jnp.float32)]
    if with_g:
        out_specs.append(pl.BlockSpec((BLK_N, HID), lambda i: (i, 0)))
        out_shape.append(jax.ShapeDtypeStruct((N, HID), jnp.float32))
    body = _update_body if with_g else (
        lambda h, p, w1, w2, b, wm, bm, ho: _update_body(
            h, p, w1, w2, b, wm, bm, ho, None)
    )
    return pl.pallas_call(
        body,
        grid=(N // BLK_N,),
        in_specs=[
            pl.BlockSpec((BLK_N, HID), lambda i: (i, 0)),
            pl.BlockSpec((NC, BLK_N, HID), lambda i: (0, i, 0)),
            pl.BlockSpec((HID, HID), lambda i: (0, 0)),
            pl.BlockSpec((HID, HID), lambda i: (0, 0)),
            pl.BlockSpec((1, HID), lambda i: (0, 0)),
            pl.BlockSpec((HID, HID), lambda i: (0, 0)),
            pl.BlockSpec((1, HID), lambda i: (0, 0)),
        ],
        out_specs=out_specs,
        out_shape=out_shape,
    )


_update_call = _make_update_call(True)
_update_last_call = _make_update_call(False)


_sc_mesh = plsc.VectorSubcoreMesh(core_axis_name="c", subcore_axis_name="s")


RING = 3
MAIN_ITERS = (NCHUNKS - 2) // RING


@functools.partial(
    pl.kernel,
    out_type=jax.ShapeDtypeStruct((NC, N, HID), jnp.float32),
    mesh=_sc_mesh,
    compiler_params=pltpu.CompilerParams(use_tc_tiling_on_sc=False),
    scratch_types=[
        pltpu.VMEM((RING, CHUNK), jnp.int32),
        pltpu.VMEM((RING, CHUNK), jnp.int32),
        pltpu.VMEM((RING, CHUNK, HID), jnp.float32),
        pltpu.VMEM((RING, CHUNK, HID), jnp.float32),
        pltpu.VMEM_SHARED((N, HID), jnp.float32),
        pltpu.VMEM_SHARED((N, HID), jnp.float32),
        pltpu.SemaphoreType.DMA((RING,)),
        pltpu.SemaphoreType.DMA((RING,)),
        pltpu.SemaphoreType.DMA((RING,)),
    ],
)
def _edge_pass(g_hbm, ea_hbm, src_hbm, dst_hbm, out_hbm,
               idx_s, idx_d, rows, ea_buf, g_sp, acc_sp,
               load_sem, gather_sem, scat_sem):
    c = lax.axis_index("c")
    s = lax.axis_index("s")
    row0 = s * RSTRIDE
    ebase = (c * NS + s) * EDGES_PER_TILE
    eabase = s * EDGES_PER_TILE

    def issue_loads(j, b):
        base = ebase + j * CHUNK
        erow = eabase + j * CHUNK

        @pl.when(c == 0)
        def _():
            pltpu.async_copy(ea_hbm.at[pl.ds(erow, CHUNK), pl.ds(0, HID)],
                             ea_buf.at[b], load_sem.at[b])

        @pl.when(c == 1)
        def _():
            pltpu.async_copy(ea_hbm.at[pl.ds(erow, CHUNK), pl.ds(HID, HID)],
                             ea_buf.at[b], load_sem.at[b])

        pltpu.async_copy(src_hbm.at[pl.ds(base, CHUNK)], idx_s.at[b],
                         load_sem.at[b])
        pltpu.async_copy(dst_hbm.at[pl.ds(base, CHUNK)], idx_d.at[b],
                         load_sem.at[b])

    def wait_loads(b):
        pltpu.make_async_copy(src_hbm.at[pl.ds(0, CHUNK)], idx_s.at[b],
                              load_sem.at[b]).wait()
        pltpu.make_async_copy(dst_hbm.at[pl.ds(0, CHUNK)], idx_d.at[b],
                              load_sem.at[b]).wait()
        pltpu.make_async_copy(ea_hbm.at[pl.ds(0, CHUNK), pl.ds(0, HID)],
                              ea_buf.at[b], load_sem.at[b]).wait()

    def issue_gather(b):
        pltpu.async_copy(g_sp.at[idx_s.at[b]], rows.at[b], gather_sem.at[b])

    def wait_gather(b):
        pltpu.make_async_copy(g_sp.at[idx_s.at[b]], rows.at[b],
                              gather_sem.at[b]).wait()

    def issue_scat(b):
        pltpu.async_copy(rows.at[b], acc_sp.at[idx_d.at[b]], scat_sem.at[b],
                         add=True)

    def wait_scat(b):
        pltpu.make_async_copy(rows.at[b], acc_sp.at[idx_d.at[b]],
                              scat_sem.at[b]).wait()

    UNROLL = 8

    def compute(b):
        def relu_body(jo, inner):
            j0 = jo * UNROLL
            for jj in range(UNROLL):
                for k in range(HID // LANES):
                    sl = pl.ds(k * LANES, LANES)
                    rows[b, j0 + jj, sl] = jnp.maximum(
                        rows[b, j0 + jj, sl] + ea_buf[b, j0 + jj, sl], 0.0)
            return inner

        lax.fori_loop(0, CHUNK // UNROLL, relu_body, 0)

    issue_loads(0, 0)
    issue_loads(1, 1)
    for i in range(RSIZE // CHUNK):
        sl = pl.ds(row0 + i * CHUNK, CHUNK)
        pltpu.sync_copy(g_hbm.at[sl], rows.at[2])
        pltpu.sync_copy(rows.at[2], g_sp.at[sl])
    zero = jnp.zeros((LANES,), jnp.float32)

    def zero_body(j, carry):
        for k in range(HID // LANES):
            rows[2, j, pl.ds(k * LANES, LANES)] = zero
        return carry

    lax.fori_loop(0, CHUNK, zero_body, 0)
    for i in range(RSIZE // CHUNK):
        pltpu.sync_copy(rows.at[2], acc_sp.at[pl.ds(row0 + i * CHUNK, CHUNK)])
    plsc.subcore_barrier()
    wait_loads(0)
    issue_gather(0)

    def outer(i2, carry):
        for b in range(RING):
            j = RING * i2 + b
            bn = (b + 1) % RING
            bp = (b + 2) % RING
            if b == 0:

                @pl.when(i2 > 0)
                def _():
                    wait_scat(bp)
            else:
                wait_scat(bp)
            issue_loads(j + 2, bp)
            wait_loads(bn)
            issue_gather(bn)
            wait_gather(b)
            compute(b)
            issue_scat(b)
        return carry

    lax.fori_loop(0, MAIN_ITERS, outer, 0)

    wait_scat(2)
    wait_loads(1)
    issue_gather(1)
    wait_gather(0)
    compute(0)
    issue_scat(0)
    wait_gather(1)
    compute(1)
    issue_scat(1)
    wait_scat(0)
    wait_scat(1)
    plsc.subcore_barrier()

    for i in range(RSIZE // CHUNK):
        b = i % 2
        sl = pl.ds(row0 + i * CHUNK, CHUNK)
        pltpu.sync_copy(acc_sp.at[sl], rows.at[b])
        pltpu.sync_copy(rows.at[b], out_hbm.at[c, sl])


@jax.jit
def kernel(x, edge_index, edge_attr, W_embed, b_embed, W_msg, b_msg, W_upd,
           b_upd):
    src = edge_index[0].astype(jnp.int32)
    dst = edge_index[1].astype(jnp.int32)
    wmh = W_msg[:HID]
    wme = W_msg[HID:]
    wu1 = W_upd[:HID]
    wu2 = W_upd[HID:]
    bm = b_msg.reshape(1, HID)
    be = b_embed.reshape(1, HID)
    bu = b_upd.reshape(1, HID)

    h, g = _embed_call(x, W_embed, be, wmh, bm)
    eat = edge_attr.T
    ea = _eaproj_call(eat, eat, wme)
    for t in range(STEPS):
        parts = _edge_pass(g, ea, src, dst)
        if t < STEPS - 1:
            h, g = _update_call(h, parts, wu1, wu2, bu, wmh, bm)
        else:
            h = _update_last_call(h, parts, wu1, wu2, bu, wmh, bm)[0]
    return h


# speedup vs baseline: 2.5844x; 1.1018x over previous
import functools

import jax
import jax.numpy as jnp
from jax import lax
from jax.experimental import pallas as pl
from jax.experimental.pallas import tpu as pltpu
from jax.experimental.pallas import tpu_sc as plsc

N = 10000
E = 320000
D_FEAT = 128
D_EDGE = 16
HID = 64
STEPS = 3

NC = 2
NS = 16
RSTRIDE = 624
RSIZE = 640
EDGES_PER_TILE = E // (NC * NS)
CHUNK = 80
NCHUNKS = EDGES_PER_TILE // CHUNK

LANES = 16

_PERM = []
for _m in range(HID // 32):
    for _i in range(16):
        _PERM += [32 * _m + _i, 32 * _m + 16 + _i]
_PERM = tuple(_PERM)


BLK_N = 1000
BLK_E = 6400


def _embed_body(x_ref, we_ref, be_ref, wmh_ref, bm_ref, h_ref, g_ref):
    h = jnp.dot(x_ref[...], we_ref[...], preferred_element_type=jnp.float32)
    h = h + be_ref[...]
    h_ref[...] = h
    g_ref[...] = (
        jnp.dot(h, wmh_ref[...], preferred_element_type=jnp.float32)
        + bm_ref[...]
    )


_embed_call = pl.pallas_call(
    _embed_body,
    grid=(N // BLK_N,),
    in_specs=[
        pl.BlockSpec((BLK_N, D_FEAT), lambda i: (i, 0)),
        pl.BlockSpec((D_FEAT, HID), lambda i: (0, 0)),
        pl.BlockSpec((1, HID), lambda i: (0, 0)),
        pl.BlockSpec((HID, HID), lambda i: (0, 0)),
        pl.BlockSpec((1, HID), lambda i: (0, 0)),
    ],
    out_specs=[
        pl.BlockSpec((BLK_N, HID), lambda i: (i, 0)),
        pl.BlockSpec((BLK_N, HID), lambda i: (i, 0)),
    ],
    out_shape=[
        jax.ShapeDtypeStruct((N, HID), jnp.float32),
        jax.ShapeDtypeStruct((N, HID), jnp.float32),
    ],
)


def _eaproj_body(eat_a_ref, eat_b_ref, wme_ref, out_ref):
    dn = (((0,), (0,)), ((), ()))
    pa = lax.dot_general(eat_a_ref[...], wme_ref[...], dimension_numbers=dn,
                         preferred_element_type=jnp.float32)
    pb = lax.dot_general(eat_b_ref[...], wme_ref[...], dimension_numbers=dn,
                         preferred_element_type=jnp.float32)
    out_ref[...] = jnp.concatenate([pa, pb], axis=1)


_eaproj_call = pl.pallas_call(
    _eaproj_body,
    grid=(E // 2 // BLK_E,),
    in_specs=[
        pl.BlockSpec((D_EDGE, BLK_E), lambda i: (0, i)),
        pl.BlockSpec((D_EDGE, BLK_E), lambda i: (0, i + E // 2 // BLK_E)),
        pl.BlockSpec((D_EDGE, HID), lambda i: (0, 0)),
    ],
    out_specs=pl.BlockSpec((BLK_E, 2 * HID), lambda i: (i, 0)),
    out_shape=jax.ShapeDtypeStruct((E // 2, 2 * HID), jnp.float32),
)


def _update_body(h_ref, p_ref, wu1_ref, wu2_ref, bu_ref, wmh_ref, bm_ref,
                 h_out_ref, g_out_ref):
    agg = p_ref[0] + p_ref[1]
    h = jnp.dot(h_ref[...], wu1_ref[...], preferred_element_type=jnp.float32)
    h = h + jnp.dot(agg, wu2_ref[...], preferred_element_type=jnp.float32)
    h = jnp.maximum(h + bu_ref[...], 0.0)
    h_out_ref[...] = h
    if g_out_ref is not None:
        g_out_ref[...] = (
            jnp.dot(h, wmh_ref[...], preferred_element_type=jnp.float32)
            + bm_ref[...]
        )


def _make_update_call(with_g: bool):
    out_specs = [pl.BlockSpec((BLK_N, HID), lambda i: (i, 0))]
    out_shape = [jax.ShapeDtypeStruct((N, HID), jnp.float32)]
    if with_g:
        out_specs.append(pl.BlockSpec((BLK_N, HID), lambda i: (i, 0)))
        out_shape.append(jax.ShapeDtypeStruct((N, HID), jnp.float32))
    body = _update_body if with_g else (
        lambda h, p, w1, w2, b, wm, bm, ho: _update_body(
            h, p, w1, w2, b, wm, bm, ho, None)
    )
    return pl.pallas_call(
        body,
        grid=(N // BLK_N,),
        in_specs=[
            pl.BlockSpec((BLK_N, HID), lambda i: (i, 0)),
            pl.BlockSpec((NC, BLK_N, HID), lambda i: (0, i, 0)),
            pl.BlockSpec((HID, HID), lambda i: (0, 0)),
            pl.BlockSpec((HID, HID), lambda i: (0, 0)),
            pl.BlockSpec((1, HID), lambda i: (0, 0)),
            pl.BlockSpec((HID, HID), lambda i: (0, 0)),
            pl.BlockSpec((1, HID), lambda i: (0, 0)),
        ],
        out_specs=out_specs,
        out_shape=out_shape,
    )


_update_call = _make_update_call(True)
_update_last_call = _make_update_call(False)


_sc_mesh = plsc.VectorSubcoreMesh(core_axis_name="c", subcore_axis_name="s")


RING = 4
MAIN_ITERS = (NCHUNKS - 1) // RING


@functools.partial(
    pl.kernel,
    out_type=jax.ShapeDtypeStruct((NC, N, HID), jnp.float32),
    mesh=_sc_mesh,
    compiler_params=pltpu.CompilerParams(use_tc_tiling_on_sc=False),
    scratch_types=[
        pltpu.VMEM((RING, CHUNK), jnp.int32),
        pltpu.VMEM((RING, CHUNK), jnp.int32),
        pltpu.VMEM((RING, CHUNK, HID), jnp.float32),
        pltpu.VMEM((RING, CHUNK, HID), jnp.float32),
        pltpu.VMEM_SHARED((N, HID), jnp.float32),
        pltpu.VMEM_SHARED((N, HID), jnp.float32),
        pltpu.SemaphoreType.DMA((RING,)),
        pltpu.SemaphoreType.DMA((RING,)),
        pltpu.SemaphoreType.DMA((RING,)),
    ],
)
def _edge_pass(g_hbm, ea_hbm, src_hbm, dst_hbm, out_hbm,
               idx_s, idx_d, rows, ea_buf, g_sp, acc_sp,
               load_sem, gather_sem, scat_sem):
    c = lax.axis_index("c")
    s = lax.axis_index("s")
    row0 = s * RSTRIDE
    ebase = (c * NS + s) * EDGES_PER_TILE
    eabase = s * EDGES_PER_TILE

    def issue_loads(j, b):
        base = ebase + j * CHUNK
        erow = eabase + j * CHUNK

        @pl.when(c == 0)
        def _():
            pltpu.async_copy(ea_hbm.at[pl.ds(erow, CHUNK), pl.ds(0, HID)],
                             ea_buf.at[b], load_sem.at[b])

        @pl.when(c == 1)
        def _():
            pltpu.async_copy(ea_hbm.at[pl.ds(erow, CHUNK), pl.ds(HID, HID)],
                             ea_buf.at[b], load_sem.at[b])

        pltpu.async_copy(src_hbm.at[pl.ds(base, CHUNK)], idx_s.at[b],
                         load_sem.at[b])
        pltpu.async_copy(dst_hbm.at[pl.ds(base, CHUNK)], idx_d.at[b],
                         load_sem.at[b])

    def wait_loads(b):
        pltpu.make_async_copy(src_hbm.at[pl.ds(0, CHUNK)], idx_s.at[b],
                              load_sem.at[b]).wait()
        pltpu.make_async_copy(dst_hbm.at[pl.ds(0, CHUNK)], idx_d.at[b],
                              load_sem.at[b]).wait()
        pltpu.make_async_copy(ea_hbm.at[pl.ds(0, CHUNK), pl.ds(0, HID)],
                              ea_buf.at[b], load_sem.at[b]).wait()

    def issue_gather(b):
        pltpu.async_copy(g_sp.at[idx_s.at[b]], rows.at[b], gather_sem.at[b])

    def wait_gather(b):
        pltpu.make_async_copy(g_sp.at[idx_s.at[b]], rows.at[b],
                              gather_sem.at[b]).wait()

    def issue_scat(b):
        pltpu.async_copy(rows.at[b], acc_sp.at[idx_d.at[b]], scat_sem.at[b],
                         add=True)

    def wait_scat(b):
        pltpu.make_async_copy(rows.at[b], acc_sp.at[idx_d.at[b]],
                              scat_sem.at[b]).wait()

    UNROLL = 8

    def compute(b):
        def relu_body(jo, inner):
            j0 = jo * UNROLL
            for jj in range(UNROLL):
                for k in range(HID // LANES):
                    sl = pl.ds(k * LANES, LANES)
                    rows[b, j0 + jj, sl] = jnp.maximum(
                        rows[b, j0 + jj, sl] + ea_buf[b, j0 + jj, sl], 0.0)
            return inner

        lax.fori_loop(0, CHUNK // UNROLL, relu_body, 0)

    issue_loads(0, 0)
    issue_loads(1, 1)
    for i in range(RSIZE // CHUNK):
        sl = pl.ds(row0 + i * CHUNK, CHUNK)
        pltpu.sync_copy(g_hbm.at[sl], rows.at[2])
        pltpu.sync_copy(rows.at[2], g_sp.at[sl])
    zero = jnp.zeros((LANES,), jnp.float32)

    def zero_body(j, carry):
        for k in range(HID // LANES):
            rows[2, j, pl.ds(k * LANES, LANES)] = zero
        return carry

    lax.fori_loop(0, CHUNK, zero_body, 0)
    for i in range(RSIZE // CHUNK):
        pltpu.sync_copy(rows.at[2], acc_sp.at[pl.ds(row0 + i * CHUNK, CHUNK)])
    plsc.subcore_barrier()
    wait_loads(0)
    issue_gather(0)

    def outer(i2, carry):
        for b in range(RING):
            j = RING * i2 + b
            bn = (b + 1) % RING
            bp = (b + 2) % RING
            if b < 2:

                @pl.when(i2 > 0)
                def _():
                    wait_scat(bp)
            else:
                wait_scat(bp)
            if b == RING - 1:

                @pl.when(i2 < MAIN_ITERS - 1)
                def _():
                    issue_loads(j + 2, bp)
            else:
                issue_loads(j + 2, bp)
            wait_loads(bn)
            issue_gather(bn)
            wait_gather(b)
            compute(b)
            issue_scat(b)
        return carry

    lax.fori_loop(0, MAIN_ITERS, outer, 0)

    wait_gather(0)
    compute(0)
    issue_scat(0)
    wait_scat(2)
    wait_scat(3)
    wait_scat(0)
    plsc.subcore_barrier()

    for i in range(RSIZE // CHUNK):
        b = i % 2
        sl = pl.ds(row0 + i * CHUNK, CHUNK)
        pltpu.sync_copy(acc_sp.at[sl], rows.at[b])
        pltpu.sync_copy(rows.at[b], out_hbm.at[c, sl])


@jax.jit
def kernel(x, edge_index, edge_attr, W_embed, b_embed, W_msg, b_msg, W_upd,
           b_upd):
    src = edge_index[0].astype(jnp.int32)
    dst = edge_index[1].astype(jnp.int32)
    wmh = W_msg[:HID]
    wme = W_msg[HID:]
    wu1 = W_upd[:HID]
    wu2 = W_upd[HID:]
    bm = b_msg.reshape(1, HID)
    be = b_embed.reshape(1, HID)
    bu = b_upd.reshape(1, HID)

    h, g = _embed_call(x, W_embed, be, wmh, bm)
    eat = edge_attr.T
    ea = _eaproj_call(eat, eat, wme)
    for t in range(STEPS):
        parts = _edge_pass(g, ea, src, dst)
        if t < STEPS - 1:
            h, g = _update_call(h, parts, wu1, wu2, bu, wmh, bm)
        else:
            h = _update_last_call(h, parts, wu1, wu2, bu, wmh, bm)[0]
    return h


# speedup vs baseline: 2.6052x; 1.0080x over previous
import functools

import jax
import jax.numpy as jnp
from jax import lax
from jax.experimental import pallas as pl
from jax.experimental.pallas import tpu as pltpu
from jax.experimental.pallas import tpu_sc as plsc

N = 10000
E = 320000
D_FEAT = 128
D_EDGE = 16
HID = 64
STEPS = 3

NC = 2
NS = 16
RSTRIDE = 624
RSIZE = 640
EDGES_PER_TILE = E // (NC * NS)
CHUNK = 80
NCHUNKS = EDGES_PER_TILE // CHUNK

LANES = 16


BLK_N = 1000
BLK_E = 6400


def _embed_body(x_ref, we_ref, be_ref, wmh_ref, bm_ref, h_ref, g_ref):
    h = jnp.dot(x_ref[...], we_ref[...], preferred_element_type=jnp.float32)
    h = h + be_ref[...]
    h_ref[...] = h
    g_ref[...] = (
        jnp.dot(h, wmh_ref[...], preferred_element_type=jnp.float32)
        + bm_ref[...]
    )


_embed_call = pl.pallas_call(
    _embed_body,
    grid=(N // BLK_N,),
    in_specs=[
        pl.BlockSpec((BLK_N, D_FEAT), lambda i: (i, 0)),
        pl.BlockSpec((D_FEAT, HID), lambda i: (0, 0)),
        pl.BlockSpec((1, HID), lambda i: (0, 0)),
        pl.BlockSpec((HID, HID), lambda i: (0, 0)),
        pl.BlockSpec((1, HID), lambda i: (0, 0)),
    ],
    out_specs=[
        pl.BlockSpec((BLK_N, HID), lambda i: (i, 0)),
        pl.BlockSpec((BLK_N, HID), lambda i: (i, 0)),
    ],
    out_shape=[
        jax.ShapeDtypeStruct((N, HID), jnp.float32),
        jax.ShapeDtypeStruct((N, HID), jnp.float32),
    ],
)


def _eaproj_body(eat_a_ref, eat_b_ref, wme_ref, out_ref):
    dn = (((0,), (0,)), ((), ()))
    pa = lax.dot_general(eat_a_ref[...], wme_ref[...], dimension_numbers=dn,
                         preferred_element_type=jnp.float32)
    pb = lax.dot_general(eat_b_ref[...], wme_ref[...], dimension_numbers=dn,
                         preferred_element_type=jnp.float32)
    out_ref[...] = jnp.concatenate([pa, pb], axis=1)


_eaproj_call = pl.pallas_call(
    _eaproj_body,
    grid=(E // 2 // BLK_E,),
    in_specs=[
        pl.BlockSpec((D_EDGE, BLK_E), lambda i: (0, i)),
        pl.BlockSpec((D_EDGE, BLK_E), lambda i: (0, i + E // 2 // BLK_E)),
        pl.BlockSpec((D_EDGE, HID), lambda i: (0, 0)),
    ],
    out_specs=pl.BlockSpec((BLK_E, 2 * HID), lambda i: (i, 0)),
    out_shape=jax.ShapeDtypeStruct((E // 2, 2 * HID), jnp.float32),
)


def _update_body(h_ref, p_ref, wu1_ref, wu2_ref, bu_ref, wmh_ref, bm_ref,
                 h_out_ref, g_out_ref):
    agg = p_ref[0] + p_ref[1]
    h = jnp.dot(h_ref[...], wu1_ref[...], preferred_element_type=jnp.float32)
    h = h + jnp.dot(agg, wu2_ref[...], preferred_element_type=jnp.float32)
    h = jnp.maximum(h + bu_ref[...], 0.0)
    h_out_ref[...] = h
    if g_out_ref is not None:
        g_out_ref[...] = (
            jnp.dot(h, wmh_ref[...], preferred_element_type=jnp.float32)
            + bm_ref[...]
        )


def _make_update_call(with_g: bool):
    out_specs = [pl.BlockSpec((BLK_N, HID), lambda i: (i, 0))]
    out_shape = [jax.ShapeDtypeStruct((N, HID), jnp.float32)]
    if with_g:
        out_specs.append(pl.BlockSpec((BLK_N, HID), lambda i: (i, 0)))
        out_shape.append(jax.ShapeDtypeStruct((N, HID), jnp.float32))
    body = _update_body if with_g else (
        lambda h, p, w1, w2, b, wm, bm, ho: _update_body(
            h, p, w1, w2, b, wm, bm, ho, None)
    )
    return pl.pallas_call(
        body,
        grid=(N // BLK_N,),
        in_specs=[
            pl.BlockSpec((BLK_N, HID), lambda i: (i, 0)),
            pl.BlockSpec((NC, BLK_N, HID), lambda i: (0, i, 0)),
            pl.BlockSpec((HID, HID), lambda i: (0, 0)),
            pl.BlockSpec((HID, HID), lambda i: (0, 0)),
            pl.BlockSpec((1, HID), lambda i: (0, 0)),
            pl.BlockSpec((HID, HID), lambda i: (0, 0)),
            pl.BlockSpec((1, HID), lambda i: (0, 0)),
        ],
        out_specs=out_specs,
        out_shape=out_shape,
    )


_update_call = _make_update_call(True)
_update_last_call = _make_update_call(False)


_sc_mesh = plsc.VectorSubcoreMesh(core_axis_name="c", subcore_axis_name="s")


RING = 4
MAIN_ITERS = (NCHUNKS - 1) // RING


@functools.partial(
    pl.kernel,
    out_type=jax.ShapeDtypeStruct((NC, N, HID), jnp.float32),
    mesh=_sc_mesh,
    compiler_params=pltpu.CompilerParams(use_tc_tiling_on_sc=False),
    scratch_types=[
        pltpu.VMEM((RING, 2, CHUNK), jnp.int32),
        pltpu.VMEM((RING, CHUNK, HID), jnp.float32),
        pltpu.VMEM((RING, CHUNK, HID), jnp.float32),
        pltpu.VMEM_SHARED((N, HID), jnp.float32),
        pltpu.VMEM_SHARED((N, HID), jnp.float32),
        pltpu.SemaphoreType.DMA((RING,)),
        pltpu.SemaphoreType.DMA((RING,)),
        pltpu.SemaphoreType.DMA((RING,)),
    ],
)
def _edge_pass(g_hbm, ea_hbm, ei_hbm, out_hbm,
               idx, rows, ea_buf, g_sp, acc_sp,
               load_sem, gather_sem, scat_sem):
    c = lax.axis_index("c")
    s = lax.axis_index("s")
    row0 = s * RSTRIDE
    ebase = (c * NS + s) * EDGES_PER_TILE
    eabase = s * EDGES_PER_TILE

    def issue_loads(j, b):
        base = ebase + j * CHUNK
        erow = eabase + j * CHUNK

        @pl.when(c == 0)
        def _():
            pltpu.async_copy(ea_hbm.at[pl.ds(erow, CHUNK), pl.ds(0, HID)],
                             ea_buf.at[b], load_sem.at[b])

        @pl.when(c == 1)
        def _():
            pltpu.async_copy(ea_hbm.at[pl.ds(erow, CHUNK), pl.ds(HID, HID)],
                             ea_buf.at[b], load_sem.at[b])

        pltpu.async_copy(ei_hbm.at[:, pl.ds(base, CHUNK)], idx.at[b],
                         load_sem.at[b])

    def wait_loads(b):
        pltpu.make_async_copy(ei_hbm.at[:, pl.ds(0, CHUNK)], idx.at[b],
                              load_sem.at[b]).wait()
        pltpu.make_async_copy(ea_hbm.at[pl.ds(0, CHUNK), pl.ds(0, HID)],
                              ea_buf.at[b], load_sem.at[b]).wait()

    def issue_gather(b):
        pltpu.async_copy(g_sp.at[idx.at[b, 0]], rows.at[b], gather_sem.at[b])

    def wait_gather(b):
        pltpu.make_async_copy(g_sp.at[idx.at[b, 0]], rows.at[b],
                              gather_sem.at[b]).wait()

    def issue_scat(b):
        pltpu.async_copy(rows.at[b], acc_sp.at[idx.at[b, 1]], scat_sem.at[b],
                         add=True)

    def wait_scat(b):
        pltpu.make_async_copy(rows.at[b], acc_sp.at[idx.at[b, 1]],
                              scat_sem.at[b]).wait()

    UNROLL = 16

    def compute(b):
        def relu_body(jo, inner):
            j0 = jo * UNROLL
            for jj in range(UNROLL):
                for k in range(HID // LANES):
                    sl = pl.ds(k * LANES, LANES)
                    rows[b, j0 + jj, sl] = jnp.maximum(
                        rows[b, j0 + jj, sl] + ea_buf[b, j0 + jj, sl], 0.0)
            return inner

        lax.fori_loop(0, CHUNK // UNROLL, relu_body, 0)

    issue_loads(0, 0)
    issue_loads(1, 1)
    for i in range(RSIZE // CHUNK):
        sl = pl.ds(row0 + i * CHUNK, CHUNK)
        pltpu.sync_copy(g_hbm.at[sl], rows.at[2])
        pltpu.sync_copy(rows.at[2], g_sp.at[sl])
    zero = jnp.zeros((LANES,), jnp.float32)

    def zero_body(j, carry):
        for k in range(HID // LANES):
            rows[2, j, pl.ds(k * LANES, LANES)] = zero
        return carry

    lax.fori_loop(0, CHUNK, zero_body, 0)
    for i in range(RSIZE // CHUNK):
        pltpu.sync_copy(rows.at[2], acc_sp.at[pl.ds(row0 + i * CHUNK, CHUNK)])
    plsc.subcore_barrier()
    wait_loads(0)
    issue_gather(0)

    def outer(i2, carry):
        for b in range(RING):
            j = RING * i2 + b
            bn = (b + 1) % RING
            bp = (b + 2) % RING
            if b < 2:

                @pl.when(i2 > 0)
                def _():
                    wait_scat(bp)
            else:
                wait_scat(bp)
            if b == RING - 1:

                @pl.when(i2 < MAIN_ITERS - 1)
                def _():
                    issue_loads(j + 2, bp)
            else:
                issue_loads(j + 2, bp)
            wait_loads(bn)
            issue_gather(bn)
            wait_gather(b)
            compute(b)
            issue_scat(b)
        return carry

    lax.fori_loop(0, MAIN_ITERS, outer, 0)

    wait_gather(0)
    compute(0)
    issue_scat(0)
    wait_scat(2)
    wait_scat(3)
    wait_scat(0)
    plsc.subcore_barrier()

    for i in range(RSIZE // CHUNK):
        b = i % 2
        sl = pl.ds(row0 + i * CHUNK, CHUNK)
        pltpu.sync_copy(acc_sp.at[sl], rows.at[b])
        pltpu.sync_copy(rows.at[b], out_hbm.at[c, sl])


@jax.jit
def kernel(x, edge_index, edge_attr, W_embed, b_embed, W_msg, b_msg, W_upd,
           b_upd):
    ei = edge_index.astype(jnp.int32)
    wmh = W_msg[:HID]
    wme = W_msg[HID:]
    wu1 = W_upd[:HID]
    wu2 = W_upd[HID:]
    bm = b_msg.reshape(1, HID)
    be = b_embed.reshape(1, HID)
    bu = b_upd.reshape(1, HID)

    h, g = _embed_call(x, W_embed, be, wmh, bm)
    eat = edge_attr.T
    ea = _eaproj_call(eat, eat, wme)
    for t in range(STEPS):
        parts = _edge_pass(g, ea, ei)
        if t < STEPS - 1:
            h, g = _update_call(h, parts, wu1, wu2, bu, wmh, bm)
        else:
            h = _update_last_call(h, parts, wu1, wu2, bu, wmh, bm)[0]
    return h
